# Initial kernel scaffold; baseline (speedup 1.0000x reference)
import jax
import jax.numpy as jnp
from jax.experimental import pallas as pl


def kernel(t, z):
    raise NotImplementedError("write your pallas kernel here")


# speedup vs baseline: 4.5974x; 4.5974x over previous
import jax
import jax.numpy as jnp
from jax.experimental import pallas as pl

N = 16777216
BLOCK = 1048576


def _body(z_ref, dx_ref, dy_ref):
    x = z_ref[0, :]
    y = z_ref[1, :]
    hot = x >= 2.0
    dx_ref[...] = jnp.where(hot, -y, jnp.where(y >= 0.0, -1.0, 1.0))
    dy_ref[...] = jnp.where(hot, x + 2.0, -1.0)


def kernel(t, z):
    grid = (N // BLOCK,)
    dx, dy = pl.pallas_call(
        _body,
        grid=grid,
        in_specs=[pl.BlockSpec((2, BLOCK), lambda i: (0, i))],
        out_specs=[
            pl.BlockSpec((BLOCK,), lambda i: (i,)),
            pl.BlockSpec((BLOCK,), lambda i: (i,)),
        ],
        out_shape=[
            jax.ShapeDtypeStruct((N,), jnp.float32),
            jax.ShapeDtypeStruct((N,), jnp.float32),
        ],
    )(z)
    return (dx, dy)
